# baseline (speedup 1.0000x reference)
import functools
import math

import jax
import jax.numpy as jnp
from jax import lax
from jax.experimental import pallas as pl
from jax.experimental.pallas import tpu as pltpu
from jax.experimental.pallas import tpu_sc as plsc

F32 = jnp.float32
I32 = jnp.int32

AVG_D_LOG = math.log(33.0)

N = 10000
E = 320000
D = 128
ED = 16

NC = 2
NS = 16
NW = NC * NS
CHUNKS = 32
CSZ = 313
NPAD = CHUNKS * CSZ
CROW = 320
NPADC = CHUNKS * CROW
BSCAN = 512
NBLK = E // BSCAN
GB = 16
PKSH = 14
PKMASK = (1 << PKSH) - 1

_pallas_call = pl.pallas_call


def _mm_body(x_ref, w_ref, o_ref):
    o_ref[...] = jnp.dot(x_ref[...], w_ref[...], preferred_element_type=F32)


def _xb_matmul(x, Wb):
    return _pallas_call(
        _mm_body,
        out_shape=jax.ShapeDtypeStruct((N, D), F32),
    )(x, Wb)


def _edge_body(ea_ref, w_ref, b_ref, o_ref):
    o_ref[...] = (
        jnp.dot(ea_ref[...], w_ref[...], preferred_element_type=F32)
        + b_ref[...]
    )


def _edge_pre(edge_attr, Wc, b_pre):
    blk = 6400
    return _pallas_call(
        _edge_body,
        grid=(E // blk,),
        in_specs=[
            pl.BlockSpec((blk, ED), lambda i: (i, 0)),
            pl.BlockSpec((ED, D), lambda i: (0, 0)),
            pl.BlockSpec((1, D), lambda i: (0, 0)),
        ],
        out_specs=pl.BlockSpec((blk, D), lambda i: (i, 0)),
        out_shape=jax.ShapeDtypeStruct((E, D), F32),
    )(edge_attr, Wc, b_pre.reshape(1, D))


def _sc_body(xb_h, eaC_h, pk_h, S_h, Q_h, M_h, C_h,
             pkb, seleid, obuf, grows, xrows,
             accS, accQ, accM, accC, semd, sem1, sem2):
    wid = lax.axis_index("s") * NC + lax.axis_index("c")
    base = wid * CSZ
    iota16 = lax.iota(I32, 16)
    zeros16 = jnp.zeros((16,), F32)
    neg16 = jnp.full((16,), -3.0e38, F32)
    zeros16i = jnp.zeros((16,), I32)

    def _zi(i, _):
        seleid[pl.ds(i * 16, 16)] = zeros16i
        return 0
    lax.fori_loop(0, (BSCAN + 16) // 16, _zi, 0)

    def _za(i, _):
        s = pl.ds(i * 16, 16)
        accS[s] = zeros16
        accQ[s] = zeros16
        accM[s] = neg16
        return 0
    lax.fori_loop(0, CSZ * (D // 16), _za, 0)

    def _zc(i, _):
        accC[pl.ds(i * 16, 16)] = zeros16
        return 0
    lax.fori_loop(0, CROW // 16, _zc, 0)

    def _edg(j, _):
        o = obuf[pl.ds(j, 16)][0]
        for t in range(D // 16):
            s = pl.ds(o + t * 16, 16)
            gv = (grows[j, pl.ds(t * 16, 16)]
                  + xrows[j, pl.ds(t * 16, 16)])
            accS[s] = accS[s] + gv
            accQ[s] = accQ[s] + gv * gv
            accM[s] = jnp.maximum(accM[s], gv)
        r = lax.shift_right_logical(o, 7)
        lane = r & 15
        rb = r - lane
        accC[pl.ds(rb, 16)] = (
            accC[pl.ds(rb, 16)] + jnp.where(iota16 == lane, 1.0, 0.0))
        return 0

    def _wait_half(h):
        pltpu.make_async_copy(
            eaC_h.at[seleid.at[pl.ds(h * 16, 16)]],
            grows.at[pl.ds(h * 16, 16)], sem1).wait()
        pltpu.make_async_copy(
            eaC_h.at[seleid.at[pl.ds(h * 16, 16)]],
            xrows.at[pl.ds(h * 16, 16)], sem2).wait()

    def _drain_pend(pend):
        @pl.when(pend > 0)
        def _():
            _wait_half(0)

            @pl.when(pend > 16)
            def _():
                _wait_half(1)

            lax.fori_loop(0, pend, _edg, 0)

    pltpu.async_copy(pk_h.at[pl.ds(0, BSCAN)], pkb.at[pl.ds(0, BSCAN)], semd)

    def _blk(b, pend):
        eb = b * BSCAN
        cb = (b & 1) * BSCAN
        pltpu.make_async_copy(
            pk_h.at[pl.ds(eb, BSCAN)], pkb.at[pl.ds(cb, BSCAN)], semd).wait()

        @pl.when(b + 1 < NBLK)
        def _():
            nb_off = ((b + 1) & 1) * BSCAN
            pltpu.async_copy(
                pk_h.at[pl.ds((b + 1) * BSCAN, BSCAN)],
                pkb.at[pl.ds(nb_off, BSCAN)], semd)

        def _grp(i, cur):
            pv = pkb[pl.ds(cb + i * 16, 16)]
            dv = pv & PKMASK
            m = (dv >= base) & (dv < base + CSZ)
            k = plsc.all_reduce_population_count(m)[0]

            @pl.when(k > 0)
            def _():
                mi = jnp.where(m, 1, 0)
                csum = plsc.cumsum(mi)
                pos = (cur + csum) - mi
                ev = eb + i * 16 + iota16
                plsc.store_scatter(seleid, [pos], ev, mask=m)

            return cur + k

        cur = lax.fori_loop(0, BSCAN // 16, _grp, jnp.int32(0))

        _drain_pend(pend)

        nbat = lax.shift_right_logical(cur + 15, 4)

        def _tail(jb, _):
            evv = seleid[pl.ds(jb * 16, 16)]
            rel = (evv - eb) & (BSCAN - 1)
            pkv = plsc.load_gather(pkb, [cb + rel])
            srcv = lax.shift_right_logical(pkv, PKSH)
            obuf[pl.ds(0, 16)] = ((pkv & PKMASK) - base) * D
            c1 = pltpu.async_copy(
                eaC_h.at[seleid.at[pl.ds(jb * 16, 16)]],
                grows.at[pl.ds(0, 16)], sem1)
            c2 = pltpu.async_copy(xb_h.at[srcv], xrows.at[pl.ds(0, 16)], sem2)
            c1.wait()
            c2.wait()
            kb = jnp.minimum(cur - jb * 16, 16)
            lax.fori_loop(0, kb, _edg, 0)
            return 0

        lax.fori_loop(2, jnp.maximum(nbat, 2), _tail, 0)

        for h in range(2):
            @pl.when(cur > h * 16)
            def _(h=h):
                evv = seleid[pl.ds(h * 16, 16)]
                rel = (evv - eb) & (BSCAN - 1)
                pkv = plsc.load_gather(pkb, [cb + rel])
                srcv = lax.shift_right_logical(pkv, PKSH)
                obuf[pl.ds(h * 16, 16)] = ((pkv & PKMASK) - base) * D
                pltpu.async_copy(
                    eaC_h.at[seleid.at[pl.ds(h * 16, 16)]],
                    grows.at[pl.ds(h * 16, 16)], sem1)
                pltpu.async_copy(
                    xb_h.at[srcv], xrows.at[pl.ds(h * 16, 16)], sem2)

        return jnp.minimum(cur, 32)

    pend = lax.fori_loop(0, NBLK, _blk, jnp.int32(0))
    _drain_pend(pend)

    pltpu.sync_copy(accS, S_h.at[pl.ds(base * D, CSZ * D)])
    pltpu.sync_copy(accQ, Q_h.at[pl.ds(base * D, CSZ * D)])
    pltpu.sync_copy(accM, M_h.at[pl.ds(base * D, CSZ * D)])
    pltpu.sync_copy(accC, C_h.at[pl.ds(wid * CROW, CROW)])


def _sc_segreduce(xb, eaC, pk):
    mesh = plsc.VectorSubcoreMesh(core_axis_name="c", subcore_axis_name="s")
    fn = functools.partial(
        pl.kernel,
        out_type=[
            jax.ShapeDtypeStruct((NPAD * D,), F32),
            jax.ShapeDtypeStruct((NPAD * D,), F32),
            jax.ShapeDtypeStruct((NPAD * D,), F32),
            jax.ShapeDtypeStruct((NPADC,), F32),
        ],
        mesh=mesh,
        scratch_types=[
            pltpu.VMEM((2 * BSCAN,), I32),
            pltpu.VMEM((BSCAN + 16,), I32),
            pltpu.VMEM((48,), I32),
            pltpu.VMEM((32, D), F32),
            pltpu.VMEM((32, D), F32),
            pltpu.VMEM((CSZ * D,), F32),
            pltpu.VMEM((CSZ * D,), F32),
            pltpu.VMEM((CSZ * D,), F32),
            pltpu.VMEM((CROW,), F32),
            pltpu.SemaphoreType.DMA,
            pltpu.SemaphoreType.DMA,
            pltpu.SemaphoreType.DMA,
        ],
        compiler_params=pltpu.CompilerParams(needs_layout_passes=False),
    )(_sc_body)
    return fn(xb, eaC, pk)


def _post_body(x_ref, S_ref, Q_ref, M_ref, c_ref, sn_ref, wa_ref,
               wp_ref, bp_ref, wm_ref, bm_ref, o_ref):
    xv = x_ref[...]
    Sv = S_ref[...]
    Qv = Q_ref[...]
    Mv = M_ref[...]
    c = c_ref[...]
    cs = jnp.maximum(c, 1.0)
    xa = jnp.dot(xv, wa_ref[...], preferred_element_type=F32)
    sums = c * xa + Sv
    maxs = jnp.where(c > 0.0, xa + Mv, 0.0)
    means = sums / cs
    Sn = Sv / cs
    var = jnp.maximum(Qv / cs - Sn * Sn, 0.0)
    l_idx = jnp.log(c + 1.0)
    a1 = l_idx * (1.0 / AVG_D_LOG)
    a2 = AVG_D_LOG / jnp.maximum(l_idx, 1e-6)
    A = jnp.concatenate([sums, maxs, means, var], axis=1)
    wp = wp_ref[...]
    xo = (jnp.dot(xv, wp[0:D], preferred_element_type=F32)
          + jnp.dot(A, wp[D:5 * D], preferred_element_type=F32)
          + jnp.dot(A * a1, wp[5 * D:9 * D], preferred_element_type=F32)
          + jnp.dot(A * a2, wp[9 * D:13 * D], preferred_element_type=F32)
          + bp_ref[...])
    xo = xo * sn_ref[...]
    h = jnp.dot(xo, wm_ref[...], preferred_element_type=F32) + bm_ref[...]
    h = jnp.where(h >= 0.0, h, 0.01 * h)
    o_ref[...] = xv + h


def _post(x, S, Q, M, cnt16, snorm, Wa, W_post, b_post, W_mix, b_mix):
    blk = 1000
    g = N // blk
    return _pallas_call(
        _post_body,
        grid=(g,),
        in_specs=[
            pl.BlockSpec((blk, D), lambda i: (i, 0)),
            pl.BlockSpec((blk, D), lambda i: (i, 0)),
            pl.BlockSpec((blk, D), lambda i: (i, 0)),
            pl.BlockSpec((blk, D), lambda i: (i, 0)),
            pl.BlockSpec((blk, 1), lambda i: (i, 0)),
            pl.BlockSpec((blk, 1), lambda i: (i, 0)),
            pl.BlockSpec((D, D), lambda i: (0, 0)),
            pl.BlockSpec((13 * D, D), lambda i: (0, 0)),
            pl.BlockSpec((1, D), lambda i: (0, 0)),
            pl.BlockSpec((D, D), lambda i: (0, 0)),
            pl.BlockSpec((1, D), lambda i: (0, 0)),
        ],
        out_specs=pl.BlockSpec((blk, D), lambda i: (i, 0)),
        out_shape=jax.ShapeDtypeStruct((N, D), F32),
    )(x, S, Q, M, cnt16, snorm, Wa, W_post, b_post, W_mix, b_mix)


def kernel(x, edge_index, snorm_n, edge_attr, W_pre, b_pre, W_post, b_post,
           W_mix, b_mix):
    Wa = W_pre[:D]
    Wb = W_pre[D:2 * D]
    Wc = W_pre[2 * D:]
    xb = _xb_matmul(x, Wb)
    eaC = _edge_pre(edge_attr, Wc, b_pre)
    src = edge_index[0].astype(I32)
    dst = edge_index[1].astype(I32)
    pk = (src << PKSH) | dst
    Sf, Qf, Mf, Cf = _sc_segreduce(xb, eaC, pk)
    S = Sf.reshape(NPAD, D)[:N]
    Q = Qf.reshape(NPAD, D)[:N]
    M = Mf.reshape(NPAD, D)[:N]
    cnt = Cf.reshape(CHUNKS, CROW)[:, :CSZ].reshape(NPAD)[:N]
    return _post(x, S, Q, M, cnt.reshape(N, 1), snorm_n, Wa, W_post,
                 b_post.reshape(1, D), W_mix, b_mix.reshape(1, D))


# speedup vs baseline: 1.0522x; 1.0522x over previous
import functools
import math

import jax
import jax.numpy as jnp
from jax import lax
from jax.experimental import pallas as pl
from jax.experimental.pallas import tpu as pltpu
from jax.experimental.pallas import tpu_sc as plsc

F32 = jnp.float32
I32 = jnp.int32

AVG_D_LOG = math.log(33.0)

N = 10000
E = 320000
D = 128
ED = 16

NC = 2
NS = 16
NW = NC * NS
CHUNKS = 32
CSZ = 313
NPAD = CHUNKS * CSZ
CROW = 320
NPADC = CHUNKS * CROW
BSCAN = 512
NBLK = E // BSCAN
GB = 16
PKSH = 14
PKMASK = (1 << PKSH) - 1

_pallas_call = pl.pallas_call


def _mm_body(x_ref, w_ref, o_ref):
    o_ref[...] = jnp.dot(x_ref[...], w_ref[...], preferred_element_type=F32)


def _xb_matmul(x, Wb):
    return _pallas_call(
        _mm_body,
        out_shape=jax.ShapeDtypeStruct((N, D), F32),
    )(x, Wb)


def _edge_body(ea_ref, w_ref, b_ref, o_ref):
    o_ref[...] = (
        jnp.dot(ea_ref[...], w_ref[...], preferred_element_type=F32)
        + b_ref[...]
    )


def _edge_pre(edge_attr, Wc, b_pre):
    blk = 6400
    return _pallas_call(
        _edge_body,
        grid=(E // blk,),
        in_specs=[
            pl.BlockSpec((blk, ED), lambda i: (i, 0)),
            pl.BlockSpec((ED, D), lambda i: (0, 0)),
            pl.BlockSpec((1, D), lambda i: (0, 0)),
        ],
        out_specs=pl.BlockSpec((blk, D), lambda i: (i, 0)),
        out_shape=jax.ShapeDtypeStruct((E, D), F32),
    )(edge_attr, Wc, b_pre.reshape(1, D))


def _sc_body(xb_h, eaC_h, pk_h, S_h, Q_h, M_h, C_h,
             pkb, seleid, obuf, grows, xrows,
             accS, accQ, accM, accC, semd, sem1, sem2):
    wid = lax.axis_index("s") * NC + lax.axis_index("c")
    base = wid * CSZ
    iota16 = lax.iota(I32, 16)
    zeros16 = jnp.zeros((16,), F32)
    neg16 = jnp.full((16,), -3.0e38, F32)
    zeros16i = jnp.zeros((16,), I32)

    def _zi(i, _):
        seleid[pl.ds(i * 16, 16)] = zeros16i
        return 0
    lax.fori_loop(0, (BSCAN + 16) // 16, _zi, 0)

    def _za(i, _):
        s = pl.ds(i * 16, 16)
        accS[s] = zeros16
        accQ[s] = zeros16
        accM[s] = neg16
        return 0
    lax.fori_loop(0, CSZ * (D // 16), _za, 0)

    def _zc(i, _):
        accC[pl.ds(i * 16, 16)] = zeros16
        return 0
    lax.fori_loop(0, CROW // 16, _zc, 0)

    def _edg(j, _):
        o = obuf[pl.ds(j, 16)][0]
        for t in range(D // 16):
            s = pl.ds(o + t * 16, 16)
            gv = (grows[j, pl.ds(t * 16, 16)]
                  + xrows[j, pl.ds(t * 16, 16)])
            plsc.addupdate(accS.at[s], gv)
            plsc.addupdate(accQ.at[s], gv * gv)
            accM[s] = jnp.maximum(accM[s], gv)
        r = lax.shift_right_logical(o, 7)
        lane = r & 15
        rb = r - lane
        plsc.addupdate(accC.at[pl.ds(rb, 16)],
                       jnp.where(iota16 == lane, 1.0, 0.0))
        return 0

    def _wait_half(h):
        pltpu.make_async_copy(
            eaC_h.at[seleid.at[pl.ds(h * 16, 16)]],
            grows.at[pl.ds(h * 16, 16)], sem1).wait()
        pltpu.make_async_copy(
            eaC_h.at[seleid.at[pl.ds(h * 16, 16)]],
            xrows.at[pl.ds(h * 16, 16)], sem2).wait()

    def _drain_pend(pend):
        @pl.when(pend > 0)
        def _():
            _wait_half(0)
            lax.fori_loop(0, jnp.minimum(pend, 16), _edg, 0)

            @pl.when(pend > 16)
            def _():
                _wait_half(1)
                lax.fori_loop(16, pend, _edg, 0)

    pltpu.async_copy(pk_h.at[pl.ds(0, BSCAN)], pkb.at[pl.ds(0, BSCAN)], semd)

    def _blk(b, pend):
        eb = b * BSCAN
        cb = (b & 1) * BSCAN
        pltpu.make_async_copy(
            pk_h.at[pl.ds(eb, BSCAN)], pkb.at[pl.ds(cb, BSCAN)], semd).wait()

        @pl.when(b + 1 < NBLK)
        def _():
            nb_off = ((b + 1) & 1) * BSCAN
            pltpu.async_copy(
                pk_h.at[pl.ds((b + 1) * BSCAN, BSCAN)],
                pkb.at[pl.ds(nb_off, BSCAN)], semd)

        def _grp(i, cur):
            pv = pkb[pl.ds(cb + i * 16, 16)]
            dv = pv & PKMASK
            m = (dv >= base) & (dv < base + CSZ)
            k = plsc.all_reduce_population_count(m)[0]

            @pl.when(k > 0)
            def _():
                mi = jnp.where(m, 1, 0)
                csum = plsc.cumsum(mi)
                pos = (cur + csum) - mi
                ev = eb + i * 16 + iota16
                plsc.store_scatter(seleid, [pos], ev, mask=m)

            return cur + k

        cur = lax.fori_loop(0, BSCAN // 16, _grp, jnp.int32(0))

        _drain_pend(pend)

        nbat = lax.shift_right_logical(cur + 15, 4)

        def _tail(jb, _):
            evv = seleid[pl.ds(jb * 16, 16)]
            rel = (evv - eb) & (BSCAN - 1)
            pkv = plsc.load_gather(pkb, [cb + rel])
            srcv = lax.shift_right_logical(pkv, PKSH)
            obuf[pl.ds(0, 16)] = ((pkv & PKMASK) - base) * D
            c1 = pltpu.async_copy(
                eaC_h.at[seleid.at[pl.ds(jb * 16, 16)]],
                grows.at[pl.ds(0, 16)], sem1)
            c2 = pltpu.async_copy(xb_h.at[srcv], xrows.at[pl.ds(0, 16)], sem2)
            c1.wait()
            c2.wait()
            kb = jnp.minimum(cur - jb * 16, 16)
            lax.fori_loop(0, kb, _edg, 0)
            return 0

        lax.fori_loop(2, jnp.maximum(nbat, 2), _tail, 0)

        for h in range(2):
            @pl.when(cur > h * 16)
            def _(h=h):
                evv = seleid[pl.ds(h * 16, 16)]
                rel = (evv - eb) & (BSCAN - 1)
                pkv = plsc.load_gather(pkb, [cb + rel])
                srcv = lax.shift_right_logical(pkv, PKSH)
                obuf[pl.ds(h * 16, 16)] = ((pkv & PKMASK) - base) * D
                pltpu.async_copy(
                    eaC_h.at[seleid.at[pl.ds(h * 16, 16)]],
                    grows.at[pl.ds(h * 16, 16)], sem1)
                pltpu.async_copy(
                    xb_h.at[srcv], xrows.at[pl.ds(h * 16, 16)], sem2)

        return jnp.minimum(cur, 32)

    pend = lax.fori_loop(0, NBLK, _blk, jnp.int32(0))
    _drain_pend(pend)

    pltpu.sync_copy(accS, S_h.at[pl.ds(base * D, CSZ * D)])
    pltpu.sync_copy(accQ, Q_h.at[pl.ds(base * D, CSZ * D)])
    pltpu.sync_copy(accM, M_h.at[pl.ds(base * D, CSZ * D)])
    pltpu.sync_copy(accC, C_h.at[pl.ds(wid * CROW, CROW)])


def _sc_segreduce(xb, eaC, pk):
    mesh = plsc.VectorSubcoreMesh(core_axis_name="c", subcore_axis_name="s")
    fn = functools.partial(
        pl.kernel,
        out_type=[
            jax.ShapeDtypeStruct((NPAD * D,), F32),
            jax.ShapeDtypeStruct((NPAD * D,), F32),
            jax.ShapeDtypeStruct((NPAD * D,), F32),
            jax.ShapeDtypeStruct((NPADC,), F32),
        ],
        mesh=mesh,
        scratch_types=[
            pltpu.VMEM((2 * BSCAN,), I32),
            pltpu.VMEM((BSCAN + 16,), I32),
            pltpu.VMEM((48,), I32),
            pltpu.VMEM((32, D), F32),
            pltpu.VMEM((32, D), F32),
            pltpu.VMEM((CSZ * D,), F32),
            pltpu.VMEM((CSZ * D,), F32),
            pltpu.VMEM((CSZ * D,), F32),
            pltpu.VMEM((CROW,), F32),
            pltpu.SemaphoreType.DMA,
            pltpu.SemaphoreType.DMA,
            pltpu.SemaphoreType.DMA,
        ],
        compiler_params=pltpu.CompilerParams(needs_layout_passes=False),
    )(_sc_body)
    return fn(xb, eaC, pk)


def _post_body(x_ref, S_ref, Q_ref, M_ref, c_ref, sn_ref, wa_ref,
               wp_ref, bp_ref, wm_ref, bm_ref, o_ref):
    xv = x_ref[...]
    Sv = S_ref[...]
    Qv = Q_ref[...]
    Mv = M_ref[...]
    c = c_ref[...]
    cs = jnp.maximum(c, 1.0)
    xa = jnp.dot(xv, wa_ref[...], preferred_element_type=F32)
    sums = c * xa + Sv
    maxs = jnp.where(c > 0.0, xa + Mv, 0.0)
    means = sums / cs
    Sn = Sv / cs
    var = jnp.maximum(Qv / cs - Sn * Sn, 0.0)
    l_idx = jnp.log(c + 1.0)
    a1 = l_idx * (1.0 / AVG_D_LOG)
    a2 = AVG_D_LOG / jnp.maximum(l_idx, 1e-6)
    A = jnp.concatenate([sums, maxs, means, var], axis=1)
    wp = wp_ref[...]
    xo = (jnp.dot(xv, wp[0:D], preferred_element_type=F32)
          + jnp.dot(A, wp[D:5 * D], preferred_element_type=F32)
          + jnp.dot(A * a1, wp[5 * D:9 * D], preferred_element_type=F32)
          + jnp.dot(A * a2, wp[9 * D:13 * D], preferred_element_type=F32)
          + bp_ref[...])
    xo = xo * sn_ref[...]
    h = jnp.dot(xo, wm_ref[...], preferred_element_type=F32) + bm_ref[...]
    h = jnp.where(h >= 0.0, h, 0.01 * h)
    o_ref[...] = xv + h


def _post(x, S, Q, M, cnt16, snorm, Wa, W_post, b_post, W_mix, b_mix):
    blk = 1000
    g = N // blk
    return _pallas_call(
        _post_body,
        grid=(g,),
        in_specs=[
            pl.BlockSpec((blk, D), lambda i: (i, 0)),
            pl.BlockSpec((blk, D), lambda i: (i, 0)),
            pl.BlockSpec((blk, D), lambda i: (i, 0)),
            pl.BlockSpec((blk, D), lambda i: (i, 0)),
            pl.BlockSpec((blk, 1), lambda i: (i, 0)),
            pl.BlockSpec((blk, 1), lambda i: (i, 0)),
            pl.BlockSpec((D, D), lambda i: (0, 0)),
            pl.BlockSpec((13 * D, D), lambda i: (0, 0)),
            pl.BlockSpec((1, D), lambda i: (0, 0)),
            pl.BlockSpec((D, D), lambda i: (0, 0)),
            pl.BlockSpec((1, D), lambda i: (0, 0)),
        ],
        out_specs=pl.BlockSpec((blk, D), lambda i: (i, 0)),
        out_shape=jax.ShapeDtypeStruct((N, D), F32),
    )(x, S, Q, M, cnt16, snorm, Wa, W_post, b_post, W_mix, b_mix)


def kernel(x, edge_index, snorm_n, edge_attr, W_pre, b_pre, W_post, b_post,
           W_mix, b_mix):
    Wa = W_pre[:D]
    Wb = W_pre[D:2 * D]
    Wc = W_pre[2 * D:]
    xb = _xb_matmul(x, Wb)
    eaC = _edge_pre(edge_attr, Wc, b_pre)
    src = edge_index[0].astype(I32)
    dst = edge_index[1].astype(I32)
    pk = (src << PKSH) | dst
    Sf, Qf, Mf, Cf = _sc_segreduce(xb, eaC, pk)
    S = Sf.reshape(NPAD, D)[:N]
    Q = Qf.reshape(NPAD, D)[:N]
    M = Mf.reshape(NPAD, D)[:N]
    cnt = Cf.reshape(CHUNKS, CROW)[:, :CSZ].reshape(NPAD)[:N]
    return _post(x, S, Q, M, cnt.reshape(N, 1), snorm_n, Wa, W_post,
                 b_post.reshape(1, D), W_mix, b_mix.reshape(1, D))


# speedup vs baseline: 1.0562x; 1.0038x over previous
import functools
import math

import jax
import jax.numpy as jnp
from jax import lax
from jax.experimental import pallas as pl
from jax.experimental.pallas import tpu as pltpu
from jax.experimental.pallas import tpu_sc as plsc

F32 = jnp.float32
I32 = jnp.int32

AVG_D_LOG = math.log(33.0)

N = 10000
E = 320000
D = 128
ED = 16

NC = 2
NS = 16
NW = NC * NS
CHUNKS = 32
CSZ = 313
NPAD = CHUNKS * CSZ
CROW = 320
NPADC = CHUNKS * CROW
BSCAN = 512
NBLK = E // BSCAN
GB = 16
PKSH = 14
PKMASK = (1 << PKSH) - 1

_pallas_call = pl.pallas_call


def _mm_body(x_ref, w_ref, o_ref):
    o_ref[...] = jnp.dot(x_ref[...], w_ref[...], preferred_element_type=F32)


def _xb_matmul(x, Wb):
    return _pallas_call(
        _mm_body,
        out_shape=jax.ShapeDtypeStruct((N, D), F32),
    )(x, Wb)


def _edge_body(ea_ref, w_ref, b_ref, o_ref):
    o_ref[...] = (
        jnp.dot(ea_ref[...], w_ref[...], preferred_element_type=F32)
        + b_ref[...]
    )


def _edge_pre(edge_attr, Wc, b_pre):
    blk = 6400
    return _pallas_call(
        _edge_body,
        grid=(E // blk,),
        in_specs=[
            pl.BlockSpec((blk, ED), lambda i: (i, 0)),
            pl.BlockSpec((ED, D), lambda i: (0, 0)),
            pl.BlockSpec((1, D), lambda i: (0, 0)),
        ],
        out_specs=pl.BlockSpec((blk, D), lambda i: (i, 0)),
        out_shape=jax.ShapeDtypeStruct((E, D), F32),
    )(edge_attr, Wc, b_pre.reshape(1, D))


def _sc_body(xb_h, eaC_h, pk_h, S_h, Q_h, M_h, C_h,
             pkb, seleid, obuf, grows, xrows,
             accS, accQ, accM, accC, semd, sem1, sem2):
    wid = lax.axis_index("s") * NC + lax.axis_index("c")
    base = wid * CSZ
    iota16 = lax.iota(I32, 16)
    zeros16 = jnp.zeros((16,), F32)
    neg16 = jnp.full((16,), -3.0e38, F32)
    zeros16i = jnp.zeros((16,), I32)

    def _zi(i, _):
        seleid[pl.ds(i * 16, 16)] = zeros16i
        return 0
    lax.fori_loop(0, (BSCAN + 16) // 16, _zi, 0)

    def _za(i, _):
        s = pl.ds(i * 16, 16)
        accS[s] = zeros16
        accQ[s] = zeros16
        accM[s] = neg16
        return 0
    lax.fori_loop(0, CSZ * (D // 16), _za, 0)

    def _zc(i, _):
        accC[pl.ds(i * 16, 16)] = zeros16
        return 0
    lax.fori_loop(0, CROW // 16, _zc, 0)

    def _edg(j, _):
        o = obuf[pl.ds(j, 16)][0]
        for t in range(D // 16):
            s = pl.ds(o + t * 16, 16)
            gv = (grows[j, pl.ds(t * 16, 16)]
                  + xrows[j, pl.ds(t * 16, 16)])
            plsc.addupdate(accS.at[s], gv)
            plsc.addupdate(accQ.at[s], gv * gv)
            accM[s] = jnp.maximum(accM[s], gv)
        r = lax.shift_right_logical(o, 7)
        lane = r & 15
        rb = r - lane
        plsc.addupdate(accC.at[pl.ds(rb, 16)],
                       jnp.where(iota16 == lane, 1.0, 0.0))
        return 0

    def _wait_half(h):
        pltpu.make_async_copy(
            eaC_h.at[seleid.at[pl.ds(h * 16, 16)]],
            grows.at[pl.ds(h * 16, 16)], sem1).wait()
        pltpu.make_async_copy(
            eaC_h.at[seleid.at[pl.ds(h * 16, 16)]],
            xrows.at[pl.ds(h * 16, 16)], sem2).wait()

    def _drain_pend(pend):
        @pl.when(pend > 0)
        def _():
            _wait_half(0)
            lax.fori_loop(0, jnp.minimum(pend, 16), _edg, 0)

            @pl.when(pend > 16)
            def _():
                _wait_half(1)
                lax.fori_loop(16, pend, _edg, 0)

    pltpu.async_copy(pk_h.at[pl.ds(0, BSCAN)], pkb.at[pl.ds(0, BSCAN)], semd)

    def _blk(b, pend):
        eb = b * BSCAN
        cb = (b & 1) * BSCAN
        pltpu.make_async_copy(
            pk_h.at[pl.ds(eb, BSCAN)], pkb.at[pl.ds(cb, BSCAN)], semd).wait()

        @pl.when(b + 1 < NBLK)
        def _():
            nb_off = ((b + 1) & 1) * BSCAN
            pltpu.async_copy(
                pk_h.at[pl.ds((b + 1) * BSCAN, BSCAN)],
                pkb.at[pl.ds(nb_off, BSCAN)], semd)

        def _grp(i, cur):
            pv1 = pkb[pl.ds(cb + i * 32, 16)]
            pv2 = pkb[pl.ds(cb + i * 32 + 16, 16)]
            dv1 = pv1 & PKMASK
            dv2 = pv2 & PKMASK
            m1 = (dv1 >= base) & (dv1 < base + CSZ)
            m2 = (dv2 >= base) & (dv2 < base + CSZ)
            k1 = plsc.all_reduce_population_count(m1)[0]
            k2 = plsc.all_reduce_population_count(m2)[0]

            @pl.when(k1 > 0)
            def _():
                mi = jnp.where(m1, 1, 0)
                csum = plsc.cumsum(mi)
                pos = (cur + csum) - mi
                ev = eb + i * 32 + iota16
                plsc.store_scatter(seleid, [pos], ev, mask=m1)

            @pl.when(k2 > 0)
            def _():
                mi = jnp.where(m2, 1, 0)
                csum = plsc.cumsum(mi)
                pos = (cur + k1 + csum) - mi
                ev = eb + i * 32 + 16 + iota16
                plsc.store_scatter(seleid, [pos], ev, mask=m2)

            return cur + k1 + k2

        cur = lax.fori_loop(0, BSCAN // 32, _grp, jnp.int32(0))

        _drain_pend(pend)

        nbat = lax.shift_right_logical(cur + 15, 4)

        def _tail(jb, _):
            evv = seleid[pl.ds(jb * 16, 16)]
            rel = (evv - eb) & (BSCAN - 1)
            pkv = plsc.load_gather(pkb, [cb + rel])
            srcv = lax.shift_right_logical(pkv, PKSH)
            obuf[pl.ds(0, 16)] = ((pkv & PKMASK) - base) * D
            c1 = pltpu.async_copy(
                eaC_h.at[seleid.at[pl.ds(jb * 16, 16)]],
                grows.at[pl.ds(0, 16)], sem1)
            c2 = pltpu.async_copy(xb_h.at[srcv], xrows.at[pl.ds(0, 16)], sem2)
            c1.wait()
            c2.wait()
            kb = jnp.minimum(cur - jb * 16, 16)
            lax.fori_loop(0, kb, _edg, 0)
            return 0

        lax.fori_loop(2, jnp.maximum(nbat, 2), _tail, 0)

        for h in range(2):
            @pl.when(cur > h * 16)
            def _(h=h):
                evv = seleid[pl.ds(h * 16, 16)]
                rel = (evv - eb) & (BSCAN - 1)
                pkv = plsc.load_gather(pkb, [cb + rel])
                srcv = lax.shift_right_logical(pkv, PKSH)
                obuf[pl.ds(h * 16, 16)] = ((pkv & PKMASK) - base) * D
                pltpu.async_copy(
                    eaC_h.at[seleid.at[pl.ds(h * 16, 16)]],
                    grows.at[pl.ds(h * 16, 16)], sem1)
                pltpu.async_copy(
                    xb_h.at[srcv], xrows.at[pl.ds(h * 16, 16)], sem2)

        return jnp.minimum(cur, 32)

    pend = lax.fori_loop(0, NBLK, _blk, jnp.int32(0))
    _drain_pend(pend)

    pltpu.sync_copy(accS, S_h.at[pl.ds(base * D, CSZ * D)])
    pltpu.sync_copy(accQ, Q_h.at[pl.ds(base * D, CSZ * D)])
    pltpu.sync_copy(accM, M_h.at[pl.ds(base * D, CSZ * D)])
    pltpu.sync_copy(accC, C_h.at[pl.ds(wid * CROW, CROW)])


def _sc_segreduce(xb, eaC, pk):
    mesh = plsc.VectorSubcoreMesh(core_axis_name="c", subcore_axis_name="s")
    fn = functools.partial(
        pl.kernel,
        out_type=[
            jax.ShapeDtypeStruct((NPAD * D,), F32),
            jax.ShapeDtypeStruct((NPAD * D,), F32),
            jax.ShapeDtypeStruct((NPAD * D,), F32),
            jax.ShapeDtypeStruct((NPADC,), F32),
        ],
        mesh=mesh,
        scratch_types=[
            pltpu.VMEM((2 * BSCAN,), I32),
            pltpu.VMEM((BSCAN + 16,), I32),
            pltpu.VMEM((48,), I32),
            pltpu.VMEM((32, D), F32),
            pltpu.VMEM((32, D), F32),
            pltpu.VMEM((CSZ * D,), F32),
            pltpu.VMEM((CSZ * D,), F32),
            pltpu.VMEM((CSZ * D,), F32),
            pltpu.VMEM((CROW,), F32),
            pltpu.SemaphoreType.DMA,
            pltpu.SemaphoreType.DMA,
            pltpu.SemaphoreType.DMA,
        ],
        compiler_params=pltpu.CompilerParams(needs_layout_passes=False),
    )(_sc_body)
    return fn(xb, eaC, pk)


def _post_body(x_ref, S_ref, Q_ref, M_ref, c_ref, sn_ref, wa_ref,
               wp_ref, bp_ref, wm_ref, bm_ref, o_ref):
    xv = x_ref[...]
    Sv = S_ref[...]
    Qv = Q_ref[...]
    Mv = M_ref[...]
    c = c_ref[...]
    cs = jnp.maximum(c, 1.0)
    xa = jnp.dot(xv, wa_ref[...], preferred_element_type=F32)
    sums = c * xa + Sv
    maxs = jnp.where(c > 0.0, xa + Mv, 0.0)
    means = sums / cs
    Sn = Sv / cs
    var = jnp.maximum(Qv / cs - Sn * Sn, 0.0)
    l_idx = jnp.log(c + 1.0)
    a1 = l_idx * (1.0 / AVG_D_LOG)
    a2 = AVG_D_LOG / jnp.maximum(l_idx, 1e-6)
    A = jnp.concatenate([sums, maxs, means, var], axis=1)
    wp = wp_ref[...]
    xo = (jnp.dot(xv, wp[0:D], preferred_element_type=F32)
          + jnp.dot(A, wp[D:5 * D], preferred_element_type=F32)
          + jnp.dot(A * a1, wp[5 * D:9 * D], preferred_element_type=F32)
          + jnp.dot(A * a2, wp[9 * D:13 * D], preferred_element_type=F32)
          + bp_ref[...])
    xo = xo * sn_ref[...]
    h = jnp.dot(xo, wm_ref[...], preferred_element_type=F32) + bm_ref[...]
    h = jnp.where(h >= 0.0, h, 0.01 * h)
    o_ref[...] = xv + h


def _post(x, S, Q, M, cnt16, snorm, Wa, W_post, b_post, W_mix, b_mix):
    blk = 1000
    g = N // blk
    return _pallas_call(
        _post_body,
        grid=(g,),
        in_specs=[
            pl.BlockSpec((blk, D), lambda i: (i, 0)),
            pl.BlockSpec((blk, D), lambda i: (i, 0)),
            pl.BlockSpec((blk, D), lambda i: (i, 0)),
            pl.BlockSpec((blk, D), lambda i: (i, 0)),
            pl.BlockSpec((blk, 1), lambda i: (i, 0)),
            pl.BlockSpec((blk, 1), lambda i: (i, 0)),
            pl.BlockSpec((D, D), lambda i: (0, 0)),
            pl.BlockSpec((13 * D, D), lambda i: (0, 0)),
            pl.BlockSpec((1, D), lambda i: (0, 0)),
            pl.BlockSpec((D, D), lambda i: (0, 0)),
            pl.BlockSpec((1, D), lambda i: (0, 0)),
        ],
        out_specs=pl.BlockSpec((blk, D), lambda i: (i, 0)),
        out_shape=jax.ShapeDtypeStruct((N, D), F32),
    )(x, S, Q, M, cnt16, snorm, Wa, W_post, b_post, W_mix, b_mix)


def kernel(x, edge_index, snorm_n, edge_attr, W_pre, b_pre, W_post, b_post,
           W_mix, b_mix):
    Wa = W_pre[:D]
    Wb = W_pre[D:2 * D]
    Wc = W_pre[2 * D:]
    xb = _xb_matmul(x, Wb)
    eaC = _edge_pre(edge_attr, Wc, b_pre)
    src = edge_index[0].astype(I32)
    dst = edge_index[1].astype(I32)
    pk = (src << PKSH) | dst
    Sf, Qf, Mf, Cf = _sc_segreduce(xb, eaC, pk)
    S = Sf.reshape(NPAD, D)[:N]
    Q = Qf.reshape(NPAD, D)[:N]
    M = Mf.reshape(NPAD, D)[:N]
    cnt = Cf.reshape(CHUNKS, CROW)[:, :CSZ].reshape(NPAD)[:N]
    return _post(x, S, Q, M, cnt.reshape(N, 1), snorm_n, Wa, W_post,
                 b_post.reshape(1, D), W_mix, b_mix.reshape(1, D))


# speedup vs baseline: 1.0568x; 1.0006x over previous
import functools
import math

import jax
import jax.numpy as jnp
from jax import lax
from jax.experimental import pallas as pl
from jax.experimental.pallas import tpu as pltpu
from jax.experimental.pallas import tpu_sc as plsc

F32 = jnp.float32
I32 = jnp.int32

AVG_D_LOG = math.log(33.0)

N = 10000
E = 320000
D = 128
ED = 16

NC = 2
NS = 16
NW = NC * NS
CHUNKS = 32
CSZ = 313
NPAD = CHUNKS * CSZ
CROW = 320
NPADC = CHUNKS * CROW
BSCAN = 512
NBLK = E // BSCAN
PKSH = 14
PKMASK = (1 << PKSH) - 1

_pallas_call = pl.pallas_call


def _mm_body(x_ref, w_ref, o_ref):
    o_ref[...] = jnp.dot(x_ref[...], w_ref[...], preferred_element_type=F32)


def _xb_matmul(x, Wb):
    return _pallas_call(
        _mm_body,
        out_shape=jax.ShapeDtypeStruct((N, D), F32),
    )(x, Wb)


def _edge_body(ea_ref, w_ref, b_ref, o_ref):
    o_ref[...] = (
        jnp.dot(ea_ref[...], w_ref[...], preferred_element_type=F32)
        + b_ref[...]
    )


def _edge_pre(edge_attr, Wc, b_pre):
    blk = 6400
    return _pallas_call(
        _edge_body,
        grid=(E // blk,),
        in_specs=[
            pl.BlockSpec((blk, ED), lambda i: (i, 0)),
            pl.BlockSpec((ED, D), lambda i: (0, 0)),
            pl.BlockSpec((1, D), lambda i: (0, 0)),
        ],
        out_specs=pl.BlockSpec((blk, D), lambda i: (i, 0)),
        out_shape=jax.ShapeDtypeStruct((E, D), F32),
    )(edge_attr, Wc, b_pre.reshape(1, D))


def _sc_body(xb_h, eaC_h, pk_h, S_h, Q_h, M_h, C_h,
             pkb, seleid, obuf, grows, xrows,
             accS, accQ, accM, accC, semd, sem1, sem2):
    wid = lax.axis_index("s") * NC + lax.axis_index("c")
    base = wid * CSZ
    iota16 = lax.iota(I32, 16)
    zeros16 = jnp.zeros((16,), F32)
    neg16 = jnp.full((16,), -3.0e38, F32)
    zeros16i = jnp.zeros((16,), I32)

    def _zi(i, _):
        seleid[pl.ds(i * 16, 16)] = zeros16i
        return 0
    lax.fori_loop(0, (BSCAN + 16) // 16, _zi, 0)

    def _za(i, _):
        s = pl.ds(i * 16, 16)
        accS[s] = zeros16
        accQ[s] = zeros16
        accM[s] = neg16
        return 0
    lax.fori_loop(0, CSZ * (D // 16), _za, 0)

    def _zc(i, _):
        accC[pl.ds(i * 16, 16)] = zeros16
        return 0
    lax.fori_loop(0, CROW // 16, _zc, 0)

    def _edg(j, _):
        o = obuf[pl.ds(j, 16)][0]
        for t in range(D // 16):
            s = pl.ds(o + t * 16, 16)
            gv = (grows[j, pl.ds(t * 16, 16)]
                  + xrows[j, pl.ds(t * 16, 16)])
            plsc.addupdate(accS.at[s], gv)
            plsc.addupdate(accQ.at[s], gv * gv)
            accM[s] = jnp.maximum(accM[s], gv)
        r = lax.shift_right_logical(o, 7)
        lane = r & 15
        rb = r - lane
        plsc.addupdate(accC.at[pl.ds(rb, 16)],
                       jnp.where(iota16 == lane, 1.0, 0.0))
        return 0

    def _wait_half(h):
        pltpu.make_async_copy(
            eaC_h.at[seleid.at[pl.ds(h * 16, 16)]],
            grows.at[pl.ds(h * 16, 16)], sem1).wait()
        pltpu.make_async_copy(
            eaC_h.at[seleid.at[pl.ds(h * 16, 16)]],
            xrows.at[pl.ds(h * 16, 16)], sem2).wait()

    def _drain_pend(pend):
        @pl.when(pend > 0)
        def _():
            _wait_half(0)
            lax.fori_loop(0, jnp.minimum(pend, 16), _edg, 0)

            @pl.when(pend > 16)
            def _():
                _wait_half(1)
                lax.fori_loop(16, pend, _edg, 0)

    pltpu.async_copy(pk_h.at[pl.ds(0, BSCAN)], pkb.at[pl.ds(0, BSCAN)], semd)

    def _blk(b, pend):
        eb = b * BSCAN
        cb = (b & 1) * BSCAN
        pltpu.make_async_copy(
            pk_h.at[pl.ds(eb, BSCAN)], pkb.at[pl.ds(cb, BSCAN)], semd).wait()

        @pl.when(b + 1 < NBLK)
        def _():
            nb_off = ((b + 1) & 1) * BSCAN
            pltpu.async_copy(
                pk_h.at[pl.ds((b + 1) * BSCAN, BSCAN)],
                pkb.at[pl.ds(nb_off, BSCAN)], semd)

        def _grp(i, cur):
            pv1 = pkb[pl.ds(cb + i * 32, 16)]
            pv2 = pkb[pl.ds(cb + i * 32 + 16, 16)]
            dv1 = pv1 & PKMASK
            dv2 = pv2 & PKMASK
            m1 = (dv1 >= base) & (dv1 < base + CSZ)
            m2 = (dv2 >= base) & (dv2 < base + CSZ)
            k1 = plsc.all_reduce_population_count(m1)[0]
            k2 = plsc.all_reduce_population_count(m2)[0]

            @pl.when(k1 > 0)
            def _():
                mi = jnp.where(m1, 1, 0)
                csum = plsc.cumsum(mi)
                pos = (cur + csum) - mi
                ev = eb + i * 32 + iota16
                plsc.store_scatter(seleid, [pos], ev, mask=m1)

            @pl.when(k2 > 0)
            def _():
                mi = jnp.where(m2, 1, 0)
                csum = plsc.cumsum(mi)
                pos = (cur + k1 + csum) - mi
                ev = eb + i * 32 + 16 + iota16
                plsc.store_scatter(seleid, [pos], ev, mask=m2)

            return cur + k1 + k2

        cur = lax.fori_loop(0, BSCAN // 32, _grp, jnp.int32(0))

        _drain_pend(pend)

        nbat = lax.shift_right_logical(cur + 15, 4)

        def _tail(jb, _):
            evv = seleid[pl.ds(jb * 16, 16)]
            rel = (evv - eb) & (BSCAN - 1)
            pkv = plsc.load_gather(pkb, [cb + rel])
            srcv = lax.shift_right_logical(pkv, PKSH)
            obuf[pl.ds(0, 16)] = ((pkv & PKMASK) - base) * D
            c1 = pltpu.async_copy(
                eaC_h.at[seleid.at[pl.ds(jb * 16, 16)]],
                grows.at[pl.ds(0, 16)], sem1)
            c2 = pltpu.async_copy(xb_h.at[srcv], xrows.at[pl.ds(0, 16)], sem2)
            c1.wait()
            c2.wait()
            kb = jnp.minimum(cur - jb * 16, 16)
            lax.fori_loop(0, kb, _edg, 0)
            return 0

        lax.fori_loop(2, jnp.maximum(nbat, 2), _tail, 0)

        for h in range(2):
            @pl.when(cur > h * 16)
            def _(h=h):
                evv = seleid[pl.ds(h * 16, 16)]
                rel = (evv - eb) & (BSCAN - 1)
                pkv = plsc.load_gather(pkb, [cb + rel])
                srcv = lax.shift_right_logical(pkv, PKSH)
                obuf[pl.ds(h * 16, 16)] = ((pkv & PKMASK) - base) * D
                pltpu.async_copy(
                    eaC_h.at[seleid.at[pl.ds(h * 16, 16)]],
                    grows.at[pl.ds(h * 16, 16)], sem1)
                pltpu.async_copy(
                    xb_h.at[srcv], xrows.at[pl.ds(h * 16, 16)], sem2)

        return jnp.minimum(cur, 32)

    pend = lax.fori_loop(0, NBLK, _blk, jnp.int32(0))
    _drain_pend(pend)

    pltpu.sync_copy(accS, S_h.at[pl.ds(base * D, CSZ * D)])
    pltpu.sync_copy(accQ, Q_h.at[pl.ds(base * D, CSZ * D)])
    pltpu.sync_copy(accM, M_h.at[pl.ds(base * D, CSZ * D)])
    pltpu.sync_copy(accC, C_h.at[pl.ds(wid * CROW, CROW)])


def _sc_segreduce(xb, eaC, pk):
    mesh = plsc.VectorSubcoreMesh(core_axis_name="c", subcore_axis_name="s")
    fn = functools.partial(
        pl.kernel,
        out_type=[
            jax.ShapeDtypeStruct((NPAD * D,), F32),
            jax.ShapeDtypeStruct((NPAD * D,), F32),
            jax.ShapeDtypeStruct((NPAD * D,), F32),
            jax.ShapeDtypeStruct((NPADC,), F32),
        ],
        mesh=mesh,
        scratch_types=[
            pltpu.VMEM((2 * BSCAN,), I32),
            pltpu.VMEM((BSCAN + 16,), I32),
            pltpu.VMEM((48,), I32),
            pltpu.VMEM((32, D), F32),
            pltpu.VMEM((32, D), F32),
            pltpu.VMEM((CSZ * D,), F32),
            pltpu.VMEM((CSZ * D,), F32),
            pltpu.VMEM((CSZ * D,), F32),
            pltpu.VMEM((CROW,), F32),
            pltpu.SemaphoreType.DMA,
            pltpu.SemaphoreType.DMA,
            pltpu.SemaphoreType.DMA,
        ],
        compiler_params=pltpu.CompilerParams(needs_layout_passes=False),
    )(_sc_body)
    return fn(xb, eaC, pk)


def _post_body(x_ref, S_ref, Q_ref, M_ref, c_ref, sn_ref, wa_ref,
               wp_ref, bp_ref, wm_ref, bm_ref, o_ref):
    xv = x_ref[...]
    Sv = S_ref[...]
    Qv = Q_ref[...]
    Mv = M_ref[...]
    c = c_ref[...]
    cs = jnp.maximum(c, 1.0)
    xa = jnp.dot(xv, wa_ref[...], preferred_element_type=F32)
    sums = c * xa + Sv
    maxs = jnp.where(c > 0.0, xa + Mv, 0.0)
    means = sums / cs
    Sn = Sv / cs
    var = jnp.maximum(Qv / cs - Sn * Sn, 0.0)
    l_idx = jnp.log(c + 1.0)
    a1 = l_idx * (1.0 / AVG_D_LOG)
    a2 = AVG_D_LOG / jnp.maximum(l_idx, 1e-6)
    A = jnp.concatenate([sums, maxs, means, var], axis=1)
    wp = wp_ref[...]
    xo = (jnp.dot(xv, wp[0:D], preferred_element_type=F32)
          + jnp.dot(A, wp[D:5 * D], preferred_element_type=F32)
          + jnp.dot(A * a1, wp[5 * D:9 * D], preferred_element_type=F32)
          + jnp.dot(A * a2, wp[9 * D:13 * D], preferred_element_type=F32)
          + bp_ref[...])
    xo = xo * sn_ref[...]
    h = jnp.dot(xo, wm_ref[...], preferred_element_type=F32) + bm_ref[...]
    h = jnp.where(h >= 0.0, h, 0.01 * h)
    o_ref[...] = xv + h


def _post(x, S, Q, M, cnt16, snorm, Wa, W_post, b_post, W_mix, b_mix):
    blk = 1000
    g = N // blk
    return _pallas_call(
        _post_body,
        grid=(g,),
        in_specs=[
            pl.BlockSpec((blk, D), lambda i: (i, 0)),
            pl.BlockSpec((blk, D), lambda i: (i, 0)),
            pl.BlockSpec((blk, D), lambda i: (i, 0)),
            pl.BlockSpec((blk, D), lambda i: (i, 0)),
            pl.BlockSpec((blk, 1), lambda i: (i, 0)),
            pl.BlockSpec((blk, 1), lambda i: (i, 0)),
            pl.BlockSpec((D, D), lambda i: (0, 0)),
            pl.BlockSpec((13 * D, D), lambda i: (0, 0)),
            pl.BlockSpec((1, D), lambda i: (0, 0)),
            pl.BlockSpec((D, D), lambda i: (0, 0)),
            pl.BlockSpec((1, D), lambda i: (0, 0)),
        ],
        out_specs=pl.BlockSpec((blk, D), lambda i: (i, 0)),
        out_shape=jax.ShapeDtypeStruct((N, D), F32),
    )(x, S, Q, M, cnt16, snorm, Wa, W_post, b_post, W_mix, b_mix)


def kernel(x, edge_index, snorm_n, edge_attr, W_pre, b_pre, W_post, b_post,
           W_mix, b_mix):
    Wa = W_pre[:D]
    Wb = W_pre[D:2 * D]
    Wc = W_pre[2 * D:]
    xb = _xb_matmul(x, Wb)
    eaC = _edge_pre(edge_attr, Wc, b_pre)
    src = edge_index[0].astype(I32)
    dst = edge_index[1].astype(I32)
    pk = (src << PKSH) | dst
    Sf, Qf, Mf, Cf = _sc_segreduce(xb, eaC, pk)
    S = Sf.reshape(NPAD, D)[:N]
    Q = Qf.reshape(NPAD, D)[:N]
    M = Mf.reshape(NPAD, D)[:N]
    cnt = Cf.reshape(CHUNKS, CROW)[:, :CSZ].reshape(NPAD)[:N]
    return _post(x, S, Q, M, cnt.reshape(N, 1), snorm_n, Wa, W_post,
                 b_post.reshape(1, D), W_mix, b_mix.reshape(1, D))
